# baseline (speedup 1.0000x reference)
import functools
import jax
import jax.numpy as jnp
from jax import lax
from jax.experimental import pallas as pl
from jax.experimental.pallas import tpu as pltpu
from jax.experimental.pallas import tpu_sc as plsc

N = 25000
E = 250000
D = 64
H = 4
C = 64
NC, NS, L = 2, 16, 16
NW = NC * NS

N_P = 25088
ROWS_PER_TILE = N_P // NS
EBS = 64
NBLK = 248
E_PAD = NS * EBS * NBLK
EB_PER_TILE = E_PAD // NS
EBLKS = E_PAD // EBS

_PERM = []
for _base in (0, 32):
    for _i in range(16):
        _PERM.extend([_base + _i, _base + 16 + _i])

@functools.lru_cache(maxsize=None)
def _mesh():
    return plsc.VectorSubcoreMesh(core_axis_name="c", subcore_axis_name="s",
                                  num_cores=NC, num_subcores=NS)


def _emb_gather_body(ut, mt, uid, mid, uout, mout, idx_v, rows_v, sem):
    cid = lax.axis_index("c")
    sid = lax.axis_index("s")
    wid = sid * NC + cid
    base = wid * (N_P // NW)

    def do(table, ids, out):
        def chunk(t, _):
            b = base + t * 112
            pltpu.sync_copy(ids.at[pl.ds(b, 112)], idx_v)
            pltpu.async_copy(table.at[idx_v], rows_v, sem).wait()
            pltpu.sync_copy(rows_v, out.at[pl.ds(b, 112)])
            return ()
        lax.fori_loop(0, 7, chunk, ())

    do(ut, uid, uout)
    do(mt, mid, mout)


@functools.lru_cache(maxsize=None)
def _emb_gather_kernel():
    return pl.kernel(
        _emb_gather_body,
        out_type=(jax.ShapeDtypeStruct((N_P, D), jnp.float32),
                  jax.ShapeDtypeStruct((N_P, D), jnp.float32)),
        mesh=_mesh(),
        scratch_types=[pltpu.VMEM((112,), jnp.int32),
                       pltpu.VMEM((112, D), jnp.float32),
                       pltpu.SemaphoreType.DMA],
        compiler_params=pltpu.CompilerParams(use_tc_tiling_on_sc=False, needs_layout_passes=False),
    )


DR = N_P // 16
DR_T = DR // NS
NCHUNK = DR // 112


def _edge_pass_body(xlh, xrh, edges, wvec, avec, zn, zd, accn, accd,
                    we_v, att_v, ebuf, iasrc, iadst, isct, idnm,
                    xl_rows, xr_rows, out_rows, d_rows,
                    acc, dsh, semg0, semg1, seme0, seme1, sems0, sems1):
    cid = lax.axis_index("c")
    sid = lax.axis_index("s")
    row0 = sid * ROWS_PER_TILE
    blk0 = sid * NBLK
    lane = lax.iota(jnp.int32, 16)
    zero16 = jnp.zeros((16,), jnp.float32)
    semg = [semg0, semg1]
    seme = [seme0, seme1]
    sems = [sems0, sems1]
    NG = EBS // 16

    pltpu.sync_copy(wvec, we_v)
    pltpu.sync_copy(avec, att_v)

    for hh in range(2):
        h = cid * 2 + hh
        off = h * N_P

        pltpu.sync_copy(zn, acc.at[pl.ds(row0, ROWS_PER_TILE)])
        pltpu.sync_copy(zd, dsh.at[pl.ds(sid * DR_T, DR_T)])
        plsc.subcore_barrier()

        we_r = [we_v[pl.ds(h * C + k * 16, 16)] for k in range(4)]
        at_r = [att_v[pl.ds(h * C + k * 16, 16)] for k in range(4)]

        def fire_gather(b, q, eq):
            for k in range(NG):
                iasrc[q, pl.ds(k * 16, 16)] = (
                    ebuf[eq, 0, pl.ds(k * 16, 16)] + off)
                iadst[q, pl.ds(k * 16, 16)] = (
                    ebuf[eq, 1, pl.ds(k * 16, 16)] + off)
            hb = EBS // 2
            for hf in range(2):
                pltpu.async_copy(xlh.at[iasrc.at[q, pl.ds(hf * hb, hb)]],
                                 xl_rows.at[q, pl.ds(hf * hb, hb)], semg[q])
                pltpu.async_copy(xrh.at[iadst.at[q, pl.ds(hf * hb, hb)]],
                                 xr_rows.at[q, pl.ds(hf * hb, hb)], semg[q])

        pltpu.async_copy(edges.at[blk0], ebuf.at[0], seme[0])
        pltpu.async_copy(edges.at[blk0 + 1], ebuf.at[1], seme[1])
        pltpu.make_async_copy(edges.at[blk0], ebuf.at[0], seme[0]).wait()
        fire_gather(0, 0, 0)

        def quad(i, _):
            for u in range(4):
                b = 4 * i + u
                g2 = u % 2
                eu = u
                en = (u + 1) % 4
                ef = (u + 2) % 4

                @pl.when(b < NBLK - 1)
                def _():
                    pltpu.make_async_copy(edges.at[blk0], ebuf.at[en],
                                          seme[(u + 1) % 2]).wait()
                    fire_gather(b + 1, (u + 1) % 2, en)

                @pl.when(b < NBLK - 2)
                def _():
                    pltpu.async_copy(edges.at[blk0 + b + 2], ebuf.at[ef],
                                     seme[u % 2])

                pltpu.make_async_copy(xlh.at[pl.ds(0, EBS)],
                                      xl_rows.at[g2], semg[g2]).wait()
                pltpu.make_async_copy(xrh.at[pl.ds(0, EBS)],
                                      xr_rows.at[g2], semg[g2]).wait()

                def drain():
                    pltpu.make_async_copy(zn.at[pl.ds(0, EBS)],
                                          out_rows.at[g2], sems[g2]).wait()
                    pltpu.make_async_copy(zd.at[pl.ds(0, EBS)],
                                          d_rows.at[g2], sems[g2]).wait()
                if u >= 2:
                    drain()
                else:
                    pl.when(i >= 1)(drain)

                def group(g, _):
                    ea16 = plsc.bitcast(ebuf[eu, 2, pl.ds(g * 16, 16)],
                                        jnp.float32)
                    dl16 = lax.bitwise_and(ebuf[eu, 1, pl.ds(g * 16, 16)],
                                           15)
                    for j in range(16):
                        e = g * 16 + j
                        esp = lax.broadcast(ea16[j], (16,))
                        s01 = zero16
                        s23 = zero16
                        xls = []
                        xra, xrb = plsc.unpack(
                            xr_rows[g2, e, pl.ds(0, 32)],
                            format=plsc.PackFormat.INTERLEAVED)
                        xrc, xrd = plsc.unpack(
                            xr_rows[g2, e, pl.ds(32, 32)],
                            format=plsc.PackFormat.INTERLEAVED)
                        xrs = [xra, xrb, xrc, xrd]
                        for k in range(4):
                            xlk = xl_rows[g2, e, pl.ds(k * 16, 16)]
                            xls.append(xlk)
                            m = (xlk + xrs[k]
                                 + esp * we_r[k])
                            m = jnp.maximum(m, 0.2 * m)
                            if k < 2:
                                s01 = s01 + m * at_r[k]
                            else:
                                s23 = s23 + m * at_r[k]
                        logit = jnp.sum(s01 + s23)
                        asp = jnp.exp(lax.broadcast(logit, (16,)))
                        for k in range(4):
                            out_rows[g2, e, pl.ds(k * 16, 16)] = (
                                asp * xls[k])
                        d_rows[g2, e, :] = jnp.where(
                            lane == lax.broadcast(dl16[j], (16,)), asp, 0.0)
                    return ()
                lax.fori_loop(0, NG, group, ())

                for k in range(NG):
                    v = ebuf[eu, 1, pl.ds(k * 16, 16)]
                    isct[g2, pl.ds(k * 16, 16)] = v
                    idnm[g2, pl.ds(k * 16, 16)] = lax.shift_right_logical(
                        v, 4)
                pltpu.async_copy(out_rows.at[g2], acc.at[isct.at[g2]],
                                 sems[g2], add=True)
                pltpu.async_copy(d_rows.at[g2], dsh.at[idnm.at[g2]],
                                 sems[g2], add=True)
            return ()
        lax.fori_loop(0, NBLK // 4, quad, ())

        for g2 in (0, 1):
            pltpu.make_async_copy(zn.at[pl.ds(0, EBS)],
                                  out_rows.at[g2], sems[g2]).wait()
            pltpu.make_async_copy(zd.at[pl.ds(0, EBS)],
                                  d_rows.at[g2], sems[g2]).wait()

        plsc.subcore_barrier()
        pltpu.sync_copy(acc.at[pl.ds(row0, ROWS_PER_TILE)],
                        accn.at[h, pl.ds(row0, ROWS_PER_TILE)])
        pltpu.sync_copy(dsh.at[pl.ds(sid * DR_T, DR_T)],
                        accd.at[h, pl.ds(sid * DR_T, DR_T)])
        plsc.subcore_barrier()


@functools.lru_cache(maxsize=None)
def _edge_pass_kernel():
    return pl.kernel(
        _edge_pass_body,
        out_type=(jax.ShapeDtypeStruct((H, N_P, C), jnp.float32),
                  jax.ShapeDtypeStruct((H, DR, 16), jnp.float32)),
        mesh=_mesh(),
        scratch_types=[
        pltpu.VMEM((H * C,), jnp.float32),
        pltpu.VMEM((H * C,), jnp.float32),
        pltpu.VMEM((4, 3, EBS), jnp.int32),
        pltpu.VMEM((2, EBS), jnp.int32),
        pltpu.VMEM((2, EBS), jnp.int32),
        pltpu.VMEM((2, EBS), jnp.int32),
        pltpu.VMEM((2, EBS), jnp.int32),
        pltpu.VMEM((2, EBS, C), jnp.float32),
        pltpu.VMEM((2, EBS, C), jnp.bfloat16),
        pltpu.VMEM((2, EBS, C), jnp.float32),
        pltpu.VMEM((2, EBS, 16), jnp.float32),
        pltpu.VMEM_SHARED((N_P, C), jnp.float32),
        pltpu.VMEM_SHARED((DR, 16), jnp.float32),
        pltpu.SemaphoreType.DMA,
        pltpu.SemaphoreType.DMA,
        pltpu.SemaphoreType.DMA,
        pltpu.SemaphoreType.DMA,
        pltpu.SemaphoreType.DMA,
        pltpu.SemaphoreType.DMA,
        ],
        compiler_params=pltpu.CompilerParams(use_tc_tiling_on_sc=False, needs_layout_passes=False),
    )


BLK = 512
NB = N_P // BLK


def _proj_body(x_ref, w_ref, b_ref, o_ref):
    o_ref[...] = jnp.dot(x_ref[...], w_ref[...],
                         preferred_element_type=jnp.float32) + b_ref[...]


def _proj(x, w, b):
    return pl.pallas_call(
        _proj_body,
        grid=(NB,),
        in_specs=[pl.BlockSpec((BLK, D), lambda i: (i, 0)),
                  pl.BlockSpec((D, D), lambda i: (0, 0)),
                  pl.BlockSpec((1, D), lambda i: (0, 0))],
        out_specs=pl.BlockSpec((BLK, D), lambda i: (i, 0)),
        out_shape=jax.ShapeDtypeStruct((N_P, D), jnp.float32),
    )(x, w, b.reshape(1, D))


def _heads_body(xs_ref, xd_ref, wl_ref, wr_ref, ol_ref, or_ref):
    ol_ref[0] = jnp.dot(xs_ref[...], wl_ref[0],
                        preferred_element_type=jnp.float32)
    or_ref[0] = jnp.dot(xd_ref[...], wr_ref[0],
                        preferred_element_type=jnp.float32).astype(jnp.bfloat16)


def _heads(x_src, x_dst, wl, wr, din):
    wl4 = wl.reshape(din, H, C).transpose(1, 0, 2)
    wr4 = wr.reshape(din, H, C).transpose(1, 0, 2)[:, :, jnp.array(_PERM)]
    return pl.pallas_call(
        _heads_body,
        grid=(H, NB),
        in_specs=[pl.BlockSpec((BLK, din), lambda h, i: (i, 0)),
                  pl.BlockSpec((BLK, din), lambda h, i: (i, 0)),
                  pl.BlockSpec((1, din, C), lambda h, i: (h, 0, 0)),
                  pl.BlockSpec((1, din, C), lambda h, i: (h, 0, 0))],
        out_specs=[pl.BlockSpec((1, BLK, C), lambda h, i: (h, i, 0)),
                   pl.BlockSpec((1, BLK, C), lambda h, i: (h, i, 0))],
        out_shape=[jax.ShapeDtypeStruct((H, N_P, C), jnp.float32),
                   jax.ShapeDtypeStruct((H, N_P, C), jnp.bfloat16)],
    )(x_src, x_dst, wl4, wr4)


def _post_body(concat, acc_ref, d_ref, b_ref, g_ref, bb_ref, o_ref):
    parts = [acc_ref[h] / (d_ref[h] + 1e-16) for h in range(H)]
    if concat:
        z = jnp.concatenate(parts, axis=-1)
    else:
        z = (parts[0] + parts[1] + parts[2] + parts[3]) * 0.25
    z = z + b_ref[...]
    mu = jnp.mean(z, axis=-1, keepdims=True)
    var = jnp.mean((z - mu) ** 2, axis=-1, keepdims=True)
    y = (z - mu) / jnp.sqrt(var + 1e-5) * g_ref[...] + bb_ref[...]
    o_ref[...] = jnp.maximum(y, 0.2 * y)


def _post(acc, den, b, g, bb, concat):
    nd = H * C if concat else C
    return pl.pallas_call(
        functools.partial(_post_body, concat),
        grid=(NB,),
        in_specs=[pl.BlockSpec((H, BLK, C), lambda i: (0, i, 0)),
                  pl.BlockSpec((H, BLK, 1), lambda i: (0, i, 0)),
                  pl.BlockSpec((1, nd), lambda i: (0, 0)),
                  pl.BlockSpec((1, nd), lambda i: (0, 0)),
                  pl.BlockSpec((1, nd), lambda i: (0, 0))],
        out_specs=pl.BlockSpec((BLK, nd), lambda i: (i, 0)),
        out_shape=jax.ShapeDtypeStruct((N_P, nd), jnp.float32),
    )(acc, den.reshape(H, N_P, 1), b.reshape(1, nd),
      g.reshape(1, nd), bb.reshape(1, nd))


def _pad_ids(ids):
    return jnp.concatenate([ids, jnp.zeros((N_P - N,), jnp.int32)])


def _pack_edges(edge_index, edge_attr):
    s = jnp.concatenate([edge_index[0], jnp.zeros((E_PAD - E,), jnp.int32)])
    d = jnp.concatenate([edge_index[1],
                         jnp.full((E_PAD - E,), N_P - 1, jnp.int32)])
    e = lax.bitcast_convert_type(
        jnp.concatenate([edge_attr.reshape(E),
                         jnp.zeros((E_PAD - E,), jnp.float32)]), jnp.int32)
    return jnp.stack([s.reshape(-1, EBS), d.reshape(-1, EBS),
                      e.reshape(-1, EBS)], axis=1)


def kernel(user_ids, movie_ids, edge_index_um, edge_index_mu, edge_attr_um,
           edge_attr_mu, user_emb, movie_emb, lin_user_W, lin_user_b,
           lin_movie_W, lin_movie_b,
           Wl0_um, Wr0_um, We0_um, att0_um, b0_um,
           Wl0_mu, Wr0_mu, We0_mu, att0_mu, b0_mu,
           ln0_user_g, ln0_user_b, ln0_movie_g, ln0_movie_b,
           Wl1_um, Wr1_um, We1_um, att1_um, b1_um,
           Wl1_mu, Wr1_mu, We1_mu, att1_mu, b1_mu,
           ln1_user_g, ln1_user_b, ln1_movie_g, ln1_movie_b):
    zn = jnp.zeros((ROWS_PER_TILE, C), jnp.float32)
    zd = jnp.zeros((DR_T, 16), jnp.float32)

    ue_r, me_r = _emb_gather_kernel()(user_emb, movie_emb,
                                      _pad_ids(user_ids), _pad_ids(movie_ids))
    xu = _proj(ue_r, lin_user_W, lin_user_b)
    xm = _proj(me_r, lin_movie_W, lin_movie_b)

    edges_um = _pack_edges(edge_index_um, edge_attr_um)
    edges_mu = _pack_edges(edge_index_mu, edge_attr_mu)

    layers = [
        (Wl0_um, Wr0_um, We0_um, att0_um, b0_um,
         Wl0_mu, Wr0_mu, We0_mu, att0_mu, b0_mu,
         ln0_user_g, ln0_user_b, ln0_movie_g, ln0_movie_b, True, D),
        (Wl1_um, Wr1_um, We1_um, att1_um, b1_um,
         Wl1_mu, Wr1_mu, We1_mu, att1_mu, b1_mu,
         ln1_user_g, ln1_user_b, ln1_movie_g, ln1_movie_b, False, H * C),
    ]
    for (wl_um, wr_um, we_um, att_um, b_um,
         wl_mu, wr_mu, we_mu, att_mu, b_mu,
         lug, lub, lmg, lmb, concat, din) in layers:
        ep = _edge_pass_kernel()
        xlh_um, xrh_um = _heads(xu, xm, wl_um, wr_um, din)
        accn_m, accd_m = ep(xlh_um.reshape(H * N_P, C),
                            xrh_um.reshape(H * N_P, C), edges_um,
                            we_um.reshape(H * C), att_um.reshape(H * C),
                            zn, zd)
        xlh_mu, xrh_mu = _heads(xm, xu, wl_mu, wr_mu, din)
        accn_u, accd_u = ep(xlh_mu.reshape(H * N_P, C),
                            xrh_mu.reshape(H * N_P, C), edges_mu,
                            we_mu.reshape(H * C), att_mu.reshape(H * C),
                            zn, zd)
        xm = _post(accn_m, accd_m, b_um, lmg, lmb, concat)
        xu = _post(accn_u, accd_u, b_mu, lug, lub, concat)

    return xu[:N], xm[:N]


# speedup vs baseline: 1.0503x; 1.0503x over previous
import functools
import jax
import jax.numpy as jnp
from jax import lax
from jax.experimental import pallas as pl
from jax.experimental.pallas import tpu as pltpu
from jax.experimental.pallas import tpu_sc as plsc

N = 25000
E = 250000
D = 64
H = 4
C = 64
NC, NS, L = 2, 16, 16
NW = NC * NS

N_P = 25088
ROWS_PER_TILE = N_P // NS
EBS = 64
NBLK = 248
E_PAD = NS * EBS * NBLK
EB_PER_TILE = E_PAD // NS
EBLKS = E_PAD // EBS

@functools.lru_cache(maxsize=None)
def _mesh():
    return plsc.VectorSubcoreMesh(core_axis_name="c", subcore_axis_name="s",
                                  num_cores=NC, num_subcores=NS)


def _emb_gather_body(ut, mt, uid, mid, uout, mout, idx_v, rows_v, sem):
    cid = lax.axis_index("c")
    sid = lax.axis_index("s")
    wid = sid * NC + cid
    base = wid * (N_P // NW)

    def do(table, ids, out):
        def chunk(t, _):
            b = base + t * 112
            pltpu.sync_copy(ids.at[pl.ds(b, 112)], idx_v)
            pltpu.async_copy(table.at[idx_v], rows_v, sem).wait()
            pltpu.sync_copy(rows_v, out.at[pl.ds(b, 112)])
            return ()
        lax.fori_loop(0, 7, chunk, ())

    do(ut, uid, uout)
    do(mt, mid, mout)


@functools.lru_cache(maxsize=None)
def _emb_gather_kernel():
    return pl.kernel(
        _emb_gather_body,
        out_type=(jax.ShapeDtypeStruct((N_P, D), jnp.float32),
                  jax.ShapeDtypeStruct((N_P, D), jnp.float32)),
        mesh=_mesh(),
        scratch_types=[pltpu.VMEM((112,), jnp.int32),
                       pltpu.VMEM((112, D), jnp.float32),
                       pltpu.SemaphoreType.DMA],
        compiler_params=pltpu.CompilerParams(use_tc_tiling_on_sc=False, needs_layout_passes=False),
    )


DR = N_P // 16
DR_T = DR // NS
NCHUNK = DR // 112


def _edge_pass_body(xlh, xrh, edges, wvec, avec, zn, zd, accn, accd,
                    we_v, att_v, ebuf, iasrc, iadst, isct, idnm,
                    xl_rows, xr_rows, out_rows, d_rows,
                    acc, dsh, semg0, semg1, seme0, seme1, sems0, sems1):
    cid = lax.axis_index("c")
    sid = lax.axis_index("s")
    row0 = sid * ROWS_PER_TILE
    blk0 = sid * NBLK
    lane = lax.iota(jnp.int32, 16)
    zero16 = jnp.zeros((16,), jnp.float32)
    semg = [semg0, semg1]
    seme = [seme0, seme1]
    sems = [sems0, sems1]
    NG = EBS // 16

    pltpu.sync_copy(wvec, we_v)
    pltpu.sync_copy(avec, att_v)

    for hh in range(2):
        h = cid * 2 + hh
        off = h * N_P

        pltpu.sync_copy(zn, acc.at[pl.ds(row0, ROWS_PER_TILE)])
        pltpu.sync_copy(zd, dsh.at[pl.ds(sid * DR_T, DR_T)])
        plsc.subcore_barrier()

        we_r = [we_v[pl.ds(h * C + k * 16, 16)] for k in range(4)]
        at_r = [att_v[pl.ds(h * C + k * 16, 16)] for k in range(4)]

        def fire_gather(b, q, eq):
            for k in range(NG):
                iasrc[q, pl.ds(k * 16, 16)] = (
                    ebuf[eq, 0, pl.ds(k * 16, 16)] + off)
                iadst[q, pl.ds(k * 16, 16)] = (
                    ebuf[eq, 1, pl.ds(k * 16, 16)] + off)
            pltpu.async_copy(xlh.at[iasrc.at[q]], xl_rows.at[q], semg[q])
            pltpu.async_copy(xrh.at[iadst.at[q]], xr_rows.at[q], semg[q])

        pltpu.async_copy(edges.at[blk0], ebuf.at[0], seme[0])
        pltpu.async_copy(edges.at[blk0 + 1], ebuf.at[1], seme[1])
        pltpu.make_async_copy(edges.at[blk0], ebuf.at[0], seme[0]).wait()
        fire_gather(0, 0, 0)

        def quad(i, _):
            for u in range(4):
                b = 4 * i + u
                g2 = u % 2
                eu = u
                en = (u + 1) % 4
                ef = (u + 2) % 4

                @pl.when(b < NBLK - 1)
                def _():
                    pltpu.make_async_copy(edges.at[blk0], ebuf.at[en],
                                          seme[(u + 1) % 2]).wait()
                    fire_gather(b + 1, (u + 1) % 2, en)

                @pl.when(b < NBLK - 2)
                def _():
                    pltpu.async_copy(edges.at[blk0 + b + 2], ebuf.at[ef],
                                     seme[u % 2])

                pltpu.make_async_copy(xlh.at[pl.ds(0, EBS)],
                                      xl_rows.at[g2], semg[g2]).wait()
                pltpu.make_async_copy(xrh.at[pl.ds(0, EBS)],
                                      xr_rows.at[g2], semg[g2]).wait()

                def drain():
                    pltpu.make_async_copy(zn.at[pl.ds(0, EBS)],
                                          out_rows.at[g2], sems[g2]).wait()
                    pltpu.make_async_copy(zd.at[pl.ds(0, EBS)],
                                          d_rows.at[g2], sems[g2]).wait()
                if u >= 2:
                    drain()
                else:
                    pl.when(i >= 1)(drain)

                def group(g, _):
                    ea16 = plsc.bitcast(ebuf[eu, 2, pl.ds(g * 16, 16)],
                                        jnp.float32)
                    dl16 = lax.bitwise_and(ebuf[eu, 1, pl.ds(g * 16, 16)],
                                           15)
                    for j in range(16):
                        e = g * 16 + j
                        esp = lax.broadcast(ea16[j], (16,))
                        s01 = zero16
                        s23 = zero16
                        xls = []
                        for k in range(4):
                            xlk = xl_rows[g2, e, pl.ds(k * 16, 16)]
                            xls.append(xlk)
                            m = (xlk + xr_rows[g2, e, pl.ds(k * 16, 16)]
                                 + esp * we_r[k])
                            m = jnp.maximum(m, 0.2 * m)
                            if k < 2:
                                s01 = s01 + m * at_r[k]
                            else:
                                s23 = s23 + m * at_r[k]
                        logit = jnp.sum(s01 + s23)
                        asp = jnp.exp(lax.broadcast(logit, (16,)))
                        for k in range(4):
                            out_rows[g2, e, pl.ds(k * 16, 16)] = (
                                asp * xls[k])
                        d_rows[g2, e, :] = jnp.where(
                            lane == lax.broadcast(dl16[j], (16,)), asp, 0.0)
                    return ()
                lax.fori_loop(0, NG, group, ())

                for k in range(NG):
                    v = ebuf[eu, 1, pl.ds(k * 16, 16)]
                    isct[g2, pl.ds(k * 16, 16)] = v
                    idnm[g2, pl.ds(k * 16, 16)] = lax.shift_right_logical(
                        v, 4)
                pltpu.async_copy(out_rows.at[g2], acc.at[isct.at[g2]],
                                 sems[g2], add=True)
                pltpu.async_copy(d_rows.at[g2], dsh.at[idnm.at[g2]],
                                 sems[g2], add=True)
            return ()
        lax.fori_loop(0, NBLK // 4, quad, ())

        for g2 in (0, 1):
            pltpu.make_async_copy(zn.at[pl.ds(0, EBS)],
                                  out_rows.at[g2], sems[g2]).wait()
            pltpu.make_async_copy(zd.at[pl.ds(0, EBS)],
                                  d_rows.at[g2], sems[g2]).wait()

        plsc.subcore_barrier()
        pltpu.sync_copy(acc.at[pl.ds(row0, ROWS_PER_TILE)],
                        accn.at[h, pl.ds(row0, ROWS_PER_TILE)])
        pltpu.sync_copy(dsh.at[pl.ds(sid * DR_T, DR_T)],
                        accd.at[h, pl.ds(sid * DR_T, DR_T)])
        plsc.subcore_barrier()


@functools.lru_cache(maxsize=None)
def _edge_pass_kernel():
    return pl.kernel(
        _edge_pass_body,
        out_type=(jax.ShapeDtypeStruct((H, N_P, C), jnp.float32),
                  jax.ShapeDtypeStruct((H, DR, 16), jnp.float32)),
        mesh=_mesh(),
        scratch_types=[
        pltpu.VMEM((H * C,), jnp.float32),
        pltpu.VMEM((H * C,), jnp.float32),
        pltpu.VMEM((4, 3, EBS), jnp.int32),
        pltpu.VMEM((2, EBS), jnp.int32),
        pltpu.VMEM((2, EBS), jnp.int32),
        pltpu.VMEM((2, EBS), jnp.int32),
        pltpu.VMEM((2, EBS), jnp.int32),
        pltpu.VMEM((2, EBS, C), jnp.float32),
        pltpu.VMEM((2, EBS, C), jnp.float32),
        pltpu.VMEM((2, EBS, C), jnp.float32),
        pltpu.VMEM((2, EBS, 16), jnp.float32),
        pltpu.VMEM_SHARED((N_P, C), jnp.float32),
        pltpu.VMEM_SHARED((DR, 16), jnp.float32),
        pltpu.SemaphoreType.DMA,
        pltpu.SemaphoreType.DMA,
        pltpu.SemaphoreType.DMA,
        pltpu.SemaphoreType.DMA,
        pltpu.SemaphoreType.DMA,
        pltpu.SemaphoreType.DMA,
        ],
        compiler_params=pltpu.CompilerParams(use_tc_tiling_on_sc=False, needs_layout_passes=False),
    )


BLK = 512
NB = N_P // BLK


def _proj_body(x_ref, w_ref, b_ref, o_ref):
    o_ref[...] = jnp.dot(x_ref[...], w_ref[...],
                         preferred_element_type=jnp.float32) + b_ref[...]


def _proj(x, w, b):
    return pl.pallas_call(
        _proj_body,
        grid=(NB,),
        in_specs=[pl.BlockSpec((BLK, D), lambda i: (i, 0)),
                  pl.BlockSpec((D, D), lambda i: (0, 0)),
                  pl.BlockSpec((1, D), lambda i: (0, 0))],
        out_specs=pl.BlockSpec((BLK, D), lambda i: (i, 0)),
        out_shape=jax.ShapeDtypeStruct((N_P, D), jnp.float32),
    )(x, w, b.reshape(1, D))


def _heads_body(xs_ref, xd_ref, wl_ref, wr_ref, ol_ref, or_ref):
    ol_ref[0] = jnp.dot(xs_ref[...], wl_ref[0],
                        preferred_element_type=jnp.float32)
    or_ref[0] = jnp.dot(xd_ref[...], wr_ref[0],
                        preferred_element_type=jnp.float32)


def _heads(x_src, x_dst, wl, wr, din):
    wl4 = wl.reshape(din, H, C).transpose(1, 0, 2)
    wr4 = wr.reshape(din, H, C).transpose(1, 0, 2)
    return pl.pallas_call(
        _heads_body,
        grid=(H, NB),
        in_specs=[pl.BlockSpec((BLK, din), lambda h, i: (i, 0)),
                  pl.BlockSpec((BLK, din), lambda h, i: (i, 0)),
                  pl.BlockSpec((1, din, C), lambda h, i: (h, 0, 0)),
                  pl.BlockSpec((1, din, C), lambda h, i: (h, 0, 0))],
        out_specs=[pl.BlockSpec((1, BLK, C), lambda h, i: (h, i, 0)),
                   pl.BlockSpec((1, BLK, C), lambda h, i: (h, i, 0))],
        out_shape=[jax.ShapeDtypeStruct((H, N_P, C), jnp.float32),
                   jax.ShapeDtypeStruct((H, N_P, C), jnp.float32)],
    )(x_src, x_dst, wl4, wr4)


def _post_body(concat, acc_ref, d_ref, b_ref, g_ref, bb_ref, o_ref):
    parts = [acc_ref[h] / (d_ref[h] + 1e-16) for h in range(H)]
    if concat:
        z = jnp.concatenate(parts, axis=-1)
    else:
        z = (parts[0] + parts[1] + parts[2] + parts[3]) * 0.25
    z = z + b_ref[...]
    mu = jnp.mean(z, axis=-1, keepdims=True)
    var = jnp.mean((z - mu) ** 2, axis=-1, keepdims=True)
    y = (z - mu) / jnp.sqrt(var + 1e-5) * g_ref[...] + bb_ref[...]
    o_ref[...] = jnp.maximum(y, 0.2 * y)


def _post(acc, den, b, g, bb, concat):
    nd = H * C if concat else C
    return pl.pallas_call(
        functools.partial(_post_body, concat),
        grid=(NB,),
        in_specs=[pl.BlockSpec((H, BLK, C), lambda i: (0, i, 0)),
                  pl.BlockSpec((H, BLK, 1), lambda i: (0, i, 0)),
                  pl.BlockSpec((1, nd), lambda i: (0, 0)),
                  pl.BlockSpec((1, nd), lambda i: (0, 0)),
                  pl.BlockSpec((1, nd), lambda i: (0, 0))],
        out_specs=pl.BlockSpec((BLK, nd), lambda i: (i, 0)),
        out_shape=jax.ShapeDtypeStruct((N_P, nd), jnp.float32),
    )(acc, den.reshape(H, N_P, 1), b.reshape(1, nd),
      g.reshape(1, nd), bb.reshape(1, nd))


def _pad_ids(ids):
    return jnp.concatenate([ids, jnp.zeros((N_P - N,), jnp.int32)])


def _pack_edges(edge_index, edge_attr):
    s = jnp.concatenate([edge_index[0], jnp.zeros((E_PAD - E,), jnp.int32)])
    d = jnp.concatenate([edge_index[1],
                         jnp.full((E_PAD - E,), N_P - 1, jnp.int32)])
    e = lax.bitcast_convert_type(
        jnp.concatenate([edge_attr.reshape(E),
                         jnp.zeros((E_PAD - E,), jnp.float32)]), jnp.int32)
    return jnp.stack([s.reshape(-1, EBS), d.reshape(-1, EBS),
                      e.reshape(-1, EBS)], axis=1)


def kernel(user_ids, movie_ids, edge_index_um, edge_index_mu, edge_attr_um,
           edge_attr_mu, user_emb, movie_emb, lin_user_W, lin_user_b,
           lin_movie_W, lin_movie_b,
           Wl0_um, Wr0_um, We0_um, att0_um, b0_um,
           Wl0_mu, Wr0_mu, We0_mu, att0_mu, b0_mu,
           ln0_user_g, ln0_user_b, ln0_movie_g, ln0_movie_b,
           Wl1_um, Wr1_um, We1_um, att1_um, b1_um,
           Wl1_mu, Wr1_mu, We1_mu, att1_mu, b1_mu,
           ln1_user_g, ln1_user_b, ln1_movie_g, ln1_movie_b):
    zn = jnp.zeros((ROWS_PER_TILE, C), jnp.float32)
    zd = jnp.zeros((DR_T, 16), jnp.float32)

    ue_r, me_r = _emb_gather_kernel()(user_emb, movie_emb,
                                      _pad_ids(user_ids), _pad_ids(movie_ids))
    xu = _proj(ue_r, lin_user_W, lin_user_b)
    xm = _proj(me_r, lin_movie_W, lin_movie_b)

    edges_um = _pack_edges(edge_index_um, edge_attr_um)
    edges_mu = _pack_edges(edge_index_mu, edge_attr_mu)

    layers = [
        (Wl0_um, Wr0_um, We0_um, att0_um, b0_um,
         Wl0_mu, Wr0_mu, We0_mu, att0_mu, b0_mu,
         ln0_user_g, ln0_user_b, ln0_movie_g, ln0_movie_b, True, D),
        (Wl1_um, Wr1_um, We1_um, att1_um, b1_um,
         Wl1_mu, Wr1_mu, We1_mu, att1_mu, b1_mu,
         ln1_user_g, ln1_user_b, ln1_movie_g, ln1_movie_b, False, H * C),
    ]
    for (wl_um, wr_um, we_um, att_um, b_um,
         wl_mu, wr_mu, we_mu, att_mu, b_mu,
         lug, lub, lmg, lmb, concat, din) in layers:
        ep = _edge_pass_kernel()
        xlh_um, xrh_um = _heads(xu, xm, wl_um, wr_um, din)
        accn_m, accd_m = ep(xlh_um.reshape(H * N_P, C),
                            xrh_um.reshape(H * N_P, C), edges_um,
                            we_um.reshape(H * C), att_um.reshape(H * C),
                            zn, zd)
        xlh_mu, xrh_mu = _heads(xm, xu, wl_mu, wr_mu, din)
        accn_u, accd_u = ep(xlh_mu.reshape(H * N_P, C),
                            xrh_mu.reshape(H * N_P, C), edges_mu,
                            we_mu.reshape(H * C), att_mu.reshape(H * C),
                            zn, zd)
        xm = _post(accn_m, accd_m, b_um, lmg, lmb, concat)
        xu = _post(accn_u, accd_u, b_mu, lug, lub, concat)

    return xu[:N], xm[:N]
